# Initial kernel scaffold; baseline (speedup 1.0000x reference)
import jax
import jax.numpy as jnp
from jax.experimental import pallas as pl


def kernel(predictions, targets_boxes, targets_labels, anchors):
    raise NotImplementedError("write your pallas kernel here")


# speedup vs baseline: 20.2811x; 20.2811x over previous
import jax
import jax.numpy as jnp
from jax import lax
from jax.experimental import pallas as pl
from jax.experimental.pallas import tpu as pltpu

_B = 8
_T = 20
_A9 = 9
_HW = 4096
_R = 288
_EPS = 1e-9


def _body(tb_ref, tl_ref, pred_ref, an_ref, out_ref):
    f32 = jnp.float32
    inv224 = f32(1.0) / f32(224.0)
    ax1 = an_ref[0] * inv224
    ay1 = an_ref[1] * inv224
    ax2 = an_ref[2] * inv224
    ay2 = an_ref[3] * inv224
    acx = 0.5 * (ax1 + ax2)
    acy = 0.5 * (ay1 + ay2)
    aw = ax2 - ax1
    ah = ay2 - ay1
    area_a = aw * ah

    def batch_body(b, carry):
        obj_t, cls_t, loc_t = carry
        pb = pred_ref[b]
        ch = [jnp.concatenate([pb[a, c] for a in range(_A9)], axis=0)
              for c in range(8)]

        best = jnp.full((_R, 128), -1.0, dtype=f32)
        mx1 = jnp.zeros((_R, 128), dtype=f32)
        my1 = jnp.zeros((_R, 128), dtype=f32)
        mx2 = jnp.zeros((_R, 128), dtype=f32)
        my2 = jnp.zeros((_R, 128), dtype=f32)
        mlab = jnp.zeros((_R, 128), dtype=f32)
        for t in range(_T):
            tx1 = tb_ref[b, 4 * t + 0] * inv224
            ty1 = tb_ref[b, 4 * t + 1] * inv224
            tx2 = tb_ref[b, 4 * t + 2] * inv224
            ty2 = tb_ref[b, 4 * t + 3] * inv224
            lab = tl_ref[b, t]
            iw = jnp.maximum(jnp.minimum(ax2, tx2) - jnp.maximum(ax1, tx1), 0.0)
            ih = jnp.maximum(jnp.minimum(ay2, ty2) - jnp.maximum(ay1, ty1), 0.0)
            inter = iw * ih
            area_t = (tx2 - tx1) * (ty2 - ty1)
            iou = inter / (area_a + area_t - inter + _EPS)
            upd = iou > best
            best = jnp.where(upd, iou, best)
            mx1 = jnp.where(upd, tx1, mx1)
            my1 = jnp.where(upd, ty1, my1)
            mx2 = jnp.where(upd, tx2, mx2)
            my2 = jnp.where(upd, ty2, my2)
            mlab = jnp.where(upd, lab, mlab)

        posf = (best > 0.5).astype(f32)
        num_pos = jnp.sum(posf)
        num_pos_i = num_pos.astype(jnp.int32)
        has_pos = (num_pos_i > 0).astype(f32)
        inv_np = f32(1.0) / jnp.maximum(num_pos, 1.0)

        x = ch[4]
        bce = jnp.maximum(x, 0.0) - x * posf + jnp.log1p(jnp.exp(-jnp.abs(x)))
        obj_pos_sum = jnp.sum(bce * posf)

        c0, c1, c2 = ch[5], ch[6], ch[7]
        m = jnp.maximum(jnp.maximum(c0, c1), c2)
        lse = m + jnp.log(jnp.exp(c0 - m) + jnp.exp(c1 - m) + jnp.exp(c2 - m))
        chosen = jnp.where(mlab <= 1.5, c0, jnp.where(mlab <= 2.5, c1, c2))
        cls_sum = jnp.sum((lse - chosen) * posf)

        gcx = 0.5 * (mx1 + mx2)
        gcy = 0.5 * (my1 + my2)
        gw = mx2 - mx1
        gh = my2 - my1
        enc = (
            (gcx - acx) / aw,
            (gcy - acy) / ah,
            jnp.log(gw / aw + 1e-6),
            jnp.log(gh / ah + 1e-6),
        )
        loc_sum = f32(0.0)
        for c in range(4):
            d = ch[c] - enc[c]
            ad = jnp.abs(d)
            sl1 = jnp.where(ad < 1.0, 0.5 * d * d, ad - 0.5)
            loc_sum = loc_sum + jnp.sum(sl1 * posf)

        negm = best < 0.4
        neg_cnt = jnp.sum(negm.astype(jnp.int32))
        negl = jnp.where(negm, bce, -1.0)
        u = lax.bitcast_convert_type(negl, jnp.int32)
        k = jnp.minimum(3 * num_pos_i, neg_cnt)

        def bit_body(i, cur):
            cand = cur | jnp.left_shift(jnp.int32(1), 30 - i)
            cnt = jnp.sum((u >= cand).astype(jnp.int32))
            return jnp.where(cnt >= k, cand, cur)

        cur = lax.fori_loop(0, 31, bit_body, jnp.int32(0))
        tau = lax.bitcast_convert_type(cur, f32)
        gtm = u > cur
        cnt_gt = jnp.sum(gtm.astype(jnp.int32))
        sum_gt = jnp.sum(jnp.where(gtm, negl, 0.0))
        tau_safe = jnp.where(k > 0, tau, 0.0)
        s_top = sum_gt + (k - cnt_gt).astype(f32) * tau_safe
        obj_neg_mean = s_top / jnp.maximum(k, 1).astype(f32)

        obj_b = (obj_pos_sum * inv_np + obj_neg_mean) * has_pos
        cls_b = cls_sum * inv_np * has_pos
        loc_b = loc_sum / jnp.maximum(4.0 * num_pos, 1.0) * has_pos
        return (obj_t + obj_b, cls_t + cls_b, loc_t + loc_b)

    obj_t, cls_t, loc_t = lax.fori_loop(
        0, _B, batch_body, (jnp.float32(0.0),) * 3)
    invB = jnp.float32(1.0 / _B)
    rows = lax.broadcasted_iota(jnp.int32, (8, 128), 0)
    outv = jnp.where(rows == 0, obj_t * invB,
                     jnp.where(rows == 1, cls_t * invB,
                               jnp.where(rows == 2, loc_t * invB, 0.0)))
    out_ref[...] = outv


def kernel(predictions, targets_boxes, targets_labels, anchors):
    pred = predictions.reshape(_B, _A9, 8, 32, 128)
    an = anchors.T.reshape(4, _R, 128)
    tb = targets_boxes.reshape(_B, 4 * _T)
    tl = targets_labels.astype(jnp.float32)
    out = pl.pallas_call(
        _body,
        out_shape=jax.ShapeDtypeStruct((8, 128), jnp.float32),
        in_specs=[
            pl.BlockSpec(memory_space=pltpu.SMEM),
            pl.BlockSpec(memory_space=pltpu.SMEM),
            pl.BlockSpec(memory_space=pltpu.VMEM),
            pl.BlockSpec(memory_space=pltpu.VMEM),
        ],
        out_specs=pl.BlockSpec(memory_space=pltpu.VMEM),
    )(tb, tl, pred, an)
    obj = out[0, 0]
    cls = out[1, 0]
    loc = out[2, 0]
    return jnp.stack([obj, cls, loc, obj + cls + loc])


# speedup vs baseline: 26.2481x; 1.2942x over previous
import jax
import jax.numpy as jnp
from jax import lax
from jax.experimental import pallas as pl
from jax.experimental.pallas import tpu as pltpu

_B = 8
_T = 20
_A9 = 9
_HW = 4096
_R = 288
_EPS = 1e-9


def _body(tb_ref, tl_ref, pred_ref, an_ref, out_ref, u_ref, k_ref):
    f32 = jnp.float32
    inv224 = f32(1.0) / f32(224.0)
    ax1 = an_ref[0] * inv224
    ay1 = an_ref[1] * inv224
    ax2 = an_ref[2] * inv224
    ay2 = an_ref[3] * inv224
    acx = 0.5 * (ax1 + ax2)
    acy = 0.5 * (ay1 + ay2)
    aw = ax2 - ax1
    ah = ay2 - ay1
    area_a = aw * ah

    def batch_body(b, carry):
        obj_t, cls_t, loc_t = carry
        pb = pred_ref[b]
        ch = [jnp.concatenate([pb[a, c] for a in range(_A9)], axis=0)
              for c in range(8)]

        best = jnp.full((_R, 128), -1.0, dtype=f32)
        mx1 = jnp.zeros((_R, 128), dtype=f32)
        my1 = jnp.zeros((_R, 128), dtype=f32)
        mx2 = jnp.zeros((_R, 128), dtype=f32)
        my2 = jnp.zeros((_R, 128), dtype=f32)
        mlab = jnp.zeros((_R, 128), dtype=f32)
        for t in range(_T):
            tx1 = tb_ref[b, 4 * t + 0] * inv224
            ty1 = tb_ref[b, 4 * t + 1] * inv224
            tx2 = tb_ref[b, 4 * t + 2] * inv224
            ty2 = tb_ref[b, 4 * t + 3] * inv224
            lab = tl_ref[b, t]
            iw = jnp.maximum(jnp.minimum(ax2, tx2) - jnp.maximum(ax1, tx1), 0.0)
            ih = jnp.maximum(jnp.minimum(ay2, ty2) - jnp.maximum(ay1, ty1), 0.0)
            inter = iw * ih
            area_t = (tx2 - tx1) * (ty2 - ty1)
            iou = inter / (area_a + area_t - inter + _EPS)
            upd = iou > best
            best = jnp.where(upd, iou, best)
            mx1 = jnp.where(upd, tx1, mx1)
            my1 = jnp.where(upd, ty1, my1)
            mx2 = jnp.where(upd, tx2, mx2)
            my2 = jnp.where(upd, ty2, my2)
            mlab = jnp.where(upd, lab, mlab)

        posf = (best > 0.5).astype(f32)
        num_pos = jnp.sum(posf)
        num_pos_i = num_pos.astype(jnp.int32)
        has_pos = (num_pos_i > 0).astype(f32)
        inv_np = f32(1.0) / jnp.maximum(num_pos, 1.0)

        x = ch[4]
        bce = jnp.maximum(x, 0.0) - x * posf + jnp.log1p(jnp.exp(-jnp.abs(x)))
        obj_pos_sum = jnp.sum(bce * posf)

        c0, c1, c2 = ch[5], ch[6], ch[7]
        m = jnp.maximum(jnp.maximum(c0, c1), c2)
        lse = m + jnp.log(jnp.exp(c0 - m) + jnp.exp(c1 - m) + jnp.exp(c2 - m))
        chosen = jnp.where(mlab <= 1.5, c0, jnp.where(mlab <= 2.5, c1, c2))
        cls_sum = jnp.sum((lse - chosen) * posf)

        gcx = 0.5 * (mx1 + mx2)
        gcy = 0.5 * (my1 + my2)
        gw = mx2 - mx1
        gh = my2 - my1
        enc = (
            (gcx - acx) / aw,
            (gcy - acy) / ah,
            jnp.log(gw / aw + 1e-6),
            jnp.log(gh / ah + 1e-6),
        )
        loc_sum = f32(0.0)
        for c in range(4):
            d = ch[c] - enc[c]
            ad = jnp.abs(d)
            sl1 = jnp.where(ad < 1.0, 0.5 * d * d, ad - 0.5)
            loc_sum = loc_sum + jnp.sum(sl1 * posf)

        negm = best < 0.4
        neg_cnt = jnp.sum(negm.astype(jnp.int32))
        negl = jnp.where(negm, bce, -1.0)
        u_ref[b] = lax.bitcast_convert_type(negl, jnp.int32)
        k_ref[b] = jnp.minimum(3 * num_pos_i, neg_cnt)

        obj_b = obj_pos_sum * inv_np * has_pos
        cls_b = cls_sum * inv_np * has_pos
        loc_b = loc_sum / jnp.maximum(4.0 * num_pos, 1.0) * has_pos
        return (obj_t + obj_b, cls_t + cls_b, loc_t + loc_b)

    obj_t, cls_t, loc_t = lax.fori_loop(
        0, _B, batch_body, (jnp.float32(0.0),) * 3)

    ks = [k_ref[b] for b in range(_B)]

    def bit_body(i, curs):
        shift = jnp.left_shift(jnp.int32(1), 30 - i)
        new = []
        for b in range(_B):
            cand = curs[b] | shift
            cnt = jnp.sum((u_ref[b] >= cand).astype(jnp.int32))
            new.append(jnp.where(cnt >= ks[b], cand, curs[b]))
        return tuple(new)

    curs = lax.fori_loop(0, 31, bit_body, (jnp.int32(0),) * _B)
    f32 = jnp.float32
    for b in range(_B):
        ub = u_ref[b]
        gtm = ub > curs[b]
        cnt_gt = jnp.sum(gtm.astype(jnp.int32))
        vb = lax.bitcast_convert_type(ub, f32)
        sum_gt = jnp.sum(jnp.where(gtm, vb, 0.0))
        tau = lax.bitcast_convert_type(curs[b], f32)
        tau_safe = jnp.where(ks[b] > 0, tau, 0.0)
        s_top = sum_gt + (ks[b] - cnt_gt).astype(f32) * tau_safe
        obj_t = obj_t + s_top / jnp.maximum(ks[b], 1).astype(f32)
    invB = jnp.float32(1.0 / _B)
    rows = lax.broadcasted_iota(jnp.int32, (8, 128), 0)
    outv = jnp.where(rows == 0, obj_t * invB,
                     jnp.where(rows == 1, cls_t * invB,
                               jnp.where(rows == 2, loc_t * invB, 0.0)))
    out_ref[...] = outv


def kernel(predictions, targets_boxes, targets_labels, anchors):
    pred = predictions.reshape(_B, _A9, 8, 32, 128)
    an = anchors.T.reshape(4, _R, 128)
    tb = targets_boxes.reshape(_B, 4 * _T)
    tl = targets_labels.astype(jnp.float32)
    out = pl.pallas_call(
        _body,
        out_shape=jax.ShapeDtypeStruct((8, 128), jnp.float32),
        in_specs=[
            pl.BlockSpec(memory_space=pltpu.SMEM),
            pl.BlockSpec(memory_space=pltpu.SMEM),
            pl.BlockSpec(memory_space=pltpu.VMEM),
            pl.BlockSpec(memory_space=pltpu.VMEM),
        ],
        out_specs=pl.BlockSpec(memory_space=pltpu.VMEM),
        scratch_shapes=[
            pltpu.VMEM((_B, _R, 128), jnp.int32),
            pltpu.SMEM((_B,), jnp.int32),
        ],
    )(tb, tl, pred, an)
    obj = out[0, 0]
    cls = out[1, 0]
    loc = out[2, 0]
    return jnp.stack([obj, cls, loc, obj + cls + loc])
